# baseline (speedup 1.0000x reference)
import functools

import jax
import jax.numpy as jnp
from jax import lax
from jax.experimental import pallas as pl
from jax.experimental.pallas import tpu as pltpu
from jax.experimental.pallas import tpu_sc as plsc

_SC_ROWS = 128
_TC_BBLK = 8


@functools.lru_cache(maxsize=None)
def _build_sc(B, T, V, nrows):
    info = plsc.get_sparse_core_info()
    NC, NS, L = info.num_cores, info.num_subcores, info.num_lanes
    NW = NC * NS
    RPW = nrows // NW

    assert nrows % NW == 0 and RPW % 2 == 0
    n_pw = RPW * T
    t_chunks = -(-T // L)

    mesh = plsc.VectorSubcoreMesh(core_axis_name="c", subcore_axis_name="s")

    @functools.partial(
        pl.kernel,
        mesh=mesh,
        out_type=jax.ShapeDtypeStruct((NW * 2 * L,), jnp.float32),
        scratch_types=[
            pltpu.VMEM((n_pw,), jnp.int32),
            pltpu.VMEM((RPW * L,), jnp.int32),
            pltpu.VMEM((T, V), jnp.float32),
            pltpu.VMEM((T, V), jnp.float32),
            pltpu.VMEM((2 * L,), jnp.float32),
            pltpu.SemaphoreType.DMA,
            pltpu.SemaphoreType.DMA,
        ],
        compiler_params=pltpu.CompilerParams(
            use_tc_tiling_on_sc=True,
            needs_layout_passes=False,
        ),
    )
    def sc_loss(in_hbm, tgt_hbm, out_hbm, tgt_v, lim_v, slab0, slab1,
                res_v, sem0, sem1):
        w = lax.axis_index("s") * NC + lax.axis_index("c")
        b0 = w * RPW
        it = lax.iota(jnp.int32, L)

        slabs = (slab0, slab1)
        sems = (sem0, sem1)

        for d in range(2):
            pltpu.async_copy(in_hbm.at[b0 + d], slabs[d], sems[d])

        pltpu.sync_copy(tgt_hbm.at[pl.ds(w * n_pw, n_pw)], tgt_v)

        def row_count(r, carry):
            nnz = jnp.zeros((L,), jnp.int32)
            for c in range(t_chunks):
                pos = c * L + it
                valid = pos < T
                tv = plsc.load_gather(tgt_v, [r * T + pos], mask=valid)
                nnz = nnz + plsc.all_reduce_population_count(
                    valid & (tv > 0))
            lim_v[pl.ds(r * L, L)] = jnp.minimum(nnz + 1, T)
            return carry

        lax.fori_loop(0, RPW, row_count, 0)

        def consume(r, sv, acc):
            lim = lim_v[pl.ds(r * L, L)]
            for c in range(t_chunks):
                pos = c * L + it
                valid = pos < T
                tgt16 = plsc.load_gather(tgt_v, [r * T + pos], mask=valid)
                m = pos < lim
                vals = plsc.load_gather(sv, [pos, tgt16], mask=m)
                acc = acc - jnp.where(m, vals, jnp.zeros((L,), jnp.float32))
            return acc

        def pair_step(k, acc):
            for d in range(2):
                r = 2 * k + d
                pltpu.make_async_copy(in_hbm.at[b0 + r], slabs[d],
                                      sems[d]).wait()
                acc = consume(r, slabs[d], acc)

                @pl.when(r + 2 < RPW)
                def _():
                    pltpu.async_copy(in_hbm.at[b0 + r + 2], slabs[d],
                                     sems[d])
            return acc

        acc = lax.fori_loop(0, RPW // 2, pair_step,
                            jnp.zeros((L,), jnp.float32))

        def mask_total(r, macc):
            return macc + lim_v[pl.ds(r * L, L)].astype(jnp.float32)

        macc = lax.fori_loop(0, RPW, mask_total,
                             jnp.zeros((L,), jnp.float32))

        res_v[pl.ds(0, L)] = acc
        res_v[pl.ds(L, L)] = macc / L
        pltpu.sync_copy(res_v, out_hbm.at[pl.ds(w * 2 * L, 2 * L)])

    return sc_loss


@functools.lru_cache(maxsize=None)
def _build_tc(B, T, V, row0, nrows, bblk):
    assert nrows % bblk == 0
    nblk = nrows // bblk

    def tc_body(in_ref, tgt_ref, out_ref):
        tgt = tgt_ref[...]
        nnz = jnp.sum((tgt > 0).astype(jnp.int32), axis=1)
        lim = jnp.minimum(nnz + 1, T)
        tmask = (
            lax.broadcasted_iota(jnp.int32, (bblk, T), 1) < lim[:, None]
        )
        vals = in_ref[...]
        onehot = (
            lax.broadcasted_iota(jnp.int32, (bblk, T, V), 2)
            == tgt[:, :, None]
        )
        g = jnp.sum(jnp.where(onehot, vals, 0.0), axis=2)
        s = jnp.sum(jnp.where(tmask, -g, 0.0))
        m = jnp.sum(tmask.astype(jnp.float32))
        lane = lax.broadcasted_iota(jnp.int32, (1, 1, 128), 2)
        out_ref[...] = jnp.where(lane == 0, s, jnp.where(lane == 1, m, 0.0))

    return pl.pallas_call(
        tc_body,
        grid=(nblk,),
        in_specs=[
            pl.BlockSpec((bblk, T, V), lambda i: (i + row0 // bblk, 0, 0)),
            pl.BlockSpec((bblk, T), lambda i: (i + row0 // bblk, 0)),
        ],
        out_specs=pl.BlockSpec((1, 1, 128), lambda i: (i, 0, 0)),
        out_shape=jax.ShapeDtypeStruct((nblk, 1, 128), jnp.float32),
    )


def kernel(input, target):
    B, T, V = input.shape
    target = target.astype(jnp.int32)

    sc_rows = _SC_ROWS
    sc_loss = _build_sc(B, T, V, sc_rows)
    tc_loss = _build_tc(B, T, V, sc_rows, B - sc_rows, _TC_BBLK)

    sc_out = sc_loss(input, target.reshape(-1))
    tc_out = tc_loss(input, target)

    L = 16
    sc_out = sc_out.reshape(-1, 2 * L)
    s = jnp.sum(sc_out[:, :L]) + jnp.sum(tc_out[:, 0, 0])
    m = jnp.sum(sc_out[:, L:]) + jnp.sum(tc_out[:, 0, 1])
    return s / m
